# baseline (speedup 1.0000x reference)
import functools

import jax
import jax.numpy as jnp
from jax import lax
from jax.experimental import pallas as pl
from jax.experimental.pallas import tpu as pltpu
from jax.experimental.pallas import tpu_sc as plsc

_CH = 800
_NBUF = 4


@functools.lru_cache(maxsize=None)
def _make_gather(B, V, D):
    info = plsc.get_sparse_core_info()
    nw = info.num_cores * info.num_subcores
    b_per_w = B // nw
    n_steps = b_per_w // _CH
    n_super = n_steps // _NBUF
    mesh = plsc.VectorSubcoreMesh(core_axis_name="c", subcore_axis_name="s")

    @functools.partial(
        pl.kernel,
        mesh=mesh,
        out_type=jax.ShapeDtypeStruct((B, D), jnp.float32),
        scratch_types=[
            *[pltpu.VMEM((_CH, D), jnp.float32) for _ in range(_NBUF)],
            *[pltpu.VMEM((_CH,), jnp.int32) for _ in range(_NBUF)],
            *[pltpu.VMEM((_CH,), jnp.int32) for _ in range(_NBUF)],
            *[pltpu.SemaphoreType.DMA for _ in range(2 * _NBUF)],
        ],
        compiler_params=pltpu.CompilerParams(use_tc_tiling_on_sc=False),
    )
    def gather_kernel(idx_hbm, table_hbm, out_hbm, *rest):
        rows = rest[:_NBUF]
        idx_c = rest[_NBUF:2 * _NBUF]
        pos_c = rest[2 * _NBUF:3 * _NBUF]
        sems_g = rest[3 * _NBUF:4 * _NBUF]
        sems_o = rest[4 * _NBUF:]
        wid = lax.axis_index("s") * info.num_cores + lax.axis_index("c")
        base = wid * b_per_w
        lane = lax.iota(jnp.int32, 16)

        def build(s, j):
            def bk(k, carry):
                pos_c[j][pl.ds(k * 16, 16)] = (
                    base + s + n_steps * (k * 16 + lane))
                return carry

            lax.fori_loop(0, _CH // 16, bk, 0)

        def gi_start(j):
            pltpu.async_copy(idx_hbm.at[pos_c[j]], idx_c[j], sems_g[j])

        def gi_wait(j):
            pltpu.make_async_copy(
                idx_hbm.at[pl.ds(0, _CH)], idx_c[j], sems_g[j]).wait()

        def g_start(j):
            pltpu.async_copy(table_hbm.at[idx_c[j]], rows[j], sems_g[j])

        def g_wait(j):
            pltpu.make_async_copy(
                out_hbm.at[pl.ds(0, _CH)], rows[j], sems_g[j]).wait()

        def o_start(j):
            pltpu.async_copy(rows[j], out_hbm.at[pos_c[j]], sems_o[j])

        def o_wait(j):
            pltpu.make_async_copy(
                rows[j], out_hbm.at[pl.ds(0, _CH)], sems_o[j]).wait()

        for j in range(_NBUF):
            build(j, j)
            gi_start(j)
        for j in range(_NBUF):
            gi_wait(j)
            g_start(j)

        def body(ss, carry):
            for j in range(_NBUF):
                g_wait(j)
                o_start(j)

            @pl.when(ss + 1 < n_super)
            def _():
                for j in range(_NBUF):
                    o_wait(j)
                    build((ss + 1) * _NBUF + j, j)
                    gi_start(j)
                for j in range(_NBUF):
                    gi_wait(j)
                    g_start(j)

            return carry

        lax.fori_loop(0, n_super, body, 0)
        for j in range(_NBUF):
            o_wait(j)

    return gather_kernel


def kernel(token_ids, weight):
    idx = token_ids.reshape(-1).astype(jnp.int32)
    out = _make_gather(idx.shape[0], weight.shape[0], weight.shape[1])(
        idx, weight
    )
    return out.reshape(*token_ids.shape, weight.shape[1])


# speedup vs baseline: 1.2477x; 1.2477x over previous
import functools

import jax
import jax.numpy as jnp
from jax import lax
from jax.experimental import pallas as pl
from jax.experimental.pallas import tpu as pltpu
from jax.experimental.pallas import tpu_sc as plsc

_CH = 800
_NBUF = 4


@functools.lru_cache(maxsize=None)
def _make_gather(B, V, D):
    info = plsc.get_sparse_core_info()
    nw = info.num_cores * info.num_subcores
    b_per_w = B // nw
    n_steps = b_per_w // _CH
    n_super = n_steps // _NBUF
    mesh = plsc.VectorSubcoreMesh(core_axis_name="c", subcore_axis_name="s")

    @functools.partial(
        pl.kernel,
        mesh=mesh,
        out_type=jax.ShapeDtypeStruct((B // _CH, _CH, D), jnp.float32),
        scratch_types=[
            pltpu.VMEM((b_per_w,), jnp.int32),
            *[pltpu.VMEM((_CH, D), jnp.float32) for _ in range(_NBUF)],
            *[pltpu.SemaphoreType.DMA for _ in range(2 * _NBUF)],
        ],
        compiler_params=pltpu.CompilerParams(use_tc_tiling_on_sc=False),
    )
    def gather_kernel(idx_hbm, table_hbm, out_hbm, idx_all, *rest):
        rows = rest[:_NBUF]
        sems_g = rest[_NBUF:2 * _NBUF]
        sems_o = rest[2 * _NBUF:]
        wid = lax.axis_index("s") * info.num_cores + lax.axis_index("c")
        base = wid * b_per_w
        chunk0 = wid * n_steps
        pltpu.sync_copy(idx_hbm.at[pl.ds(base, b_per_w)], idx_all)

        def g_start(s, j):
            pltpu.async_copy(
                table_hbm.at[idx_all.at[pl.ds(s * _CH, _CH)]],
                rows[j], sems_g[j])

        def g_wait(j):
            pltpu.make_async_copy(
                out_hbm.at[0], rows[j], sems_g[j]).wait()

        def o_start(s, j):
            pltpu.async_copy(rows[j], out_hbm.at[chunk0 + s], sems_o[j])

        def o_wait(j):
            pltpu.make_async_copy(rows[j], out_hbm.at[0], sems_o[j]).wait()

        for j in range(_NBUF):
            g_start(j, j)

        def body(ss, carry):
            for j in range(_NBUF):
                g_wait(j)
                o_start(ss * _NBUF + j, j)

            @pl.when(ss + 1 < n_super)
            def _():
                for j in range(_NBUF):
                    o_wait(j)
                    g_start((ss + 1) * _NBUF + j, j)

            return carry

        lax.fori_loop(0, n_super, body, 0)
        for j in range(_NBUF):
            o_wait(j)

    return gather_kernel


def kernel(token_ids, weight):
    idx = token_ids.reshape(-1).astype(jnp.int32)
    out = _make_gather(idx.shape[0], weight.shape[0], weight.shape[1])(
        idx, weight
    )
    return out.reshape(*token_ids.shape, weight.shape[1])


# speedup vs baseline: 1.6364x; 1.3115x over previous
import functools

import jax
import jax.numpy as jnp
from jax import lax
from jax.experimental import pallas as pl
from jax.experimental.pallas import tpu as pltpu
from jax.experimental.pallas import tpu_sc as plsc

_CH = 800
_NBUF = 4


@functools.lru_cache(maxsize=None)
def _make_gather(B, S, V, D):
    info = plsc.get_sparse_core_info()
    nw = info.num_cores * info.num_subcores
    b_per_w = B // nw
    n_steps = b_per_w // _CH
    n_super = n_steps // _NBUF
    seq_per_ch = _CH // S
    mesh = plsc.VectorSubcoreMesh(core_axis_name="c", subcore_axis_name="s")

    @functools.partial(
        pl.kernel,
        mesh=mesh,
        out_type=jax.ShapeDtypeStruct((B // S, S, D), jnp.float32),
        scratch_types=[
            pltpu.VMEM((b_per_w,), jnp.int32),
            *[pltpu.VMEM((_CH, D), jnp.float32) for _ in range(_NBUF)],
            *[pltpu.SemaphoreType.DMA for _ in range(2 * _NBUF)],
        ],
        compiler_params=pltpu.CompilerParams(use_tc_tiling_on_sc=False),
    )
    def gather_kernel(idx_hbm, table_hbm, out_hbm, idx_all, *rest):
        rows = rest[:_NBUF]
        sems_g = rest[_NBUF:2 * _NBUF]
        sems_o = rest[2 * _NBUF:]
        wid = lax.axis_index("s") * info.num_cores + lax.axis_index("c")
        base = wid * b_per_w
        seq0 = wid * (b_per_w // S)
        pltpu.sync_copy(idx_hbm.at[pl.ds(base, b_per_w)], idx_all)

        def g_start(s, j):
            pltpu.async_copy(
                table_hbm.at[idx_all.at[pl.ds(s * _CH, _CH)]],
                rows[j], sems_g[j])

        def g_wait(j):
            pltpu.make_async_copy(
                table_hbm.at[pl.ds(0, _CH)], rows[j], sems_g[j]).wait()

        def o_start(s, j):
            for t in range(seq_per_ch):
                pltpu.async_copy(
                    rows[j].at[pl.ds(t * S, S)],
                    out_hbm.at[seq0 + s * seq_per_ch + t], sems_o[j])

        def o_wait(j):
            for t in range(seq_per_ch):
                pltpu.make_async_copy(
                    rows[j].at[pl.ds(0, S)], out_hbm.at[0],
                    sems_o[j]).wait()

        for j in range(_NBUF):
            g_start(j, j)

        def body(ss, carry):
            for j in range(_NBUF):
                g_wait(j)
                o_start(ss * _NBUF + j, j)

            @pl.when(ss + 1 < n_super)
            def _():
                for j in range(_NBUF):
                    o_wait(j)
                    g_start((ss + 1) * _NBUF + j, j)

            return carry

        lax.fori_loop(0, n_super, body, 0)
        for j in range(_NBUF):
            o_wait(j)

    return gather_kernel


def kernel(token_ids, weight):
    idx = token_ids.reshape(-1).astype(jnp.int32)
    n_seq, seq_len = token_ids.shape
    out = _make_gather(idx.shape[0], seq_len, weight.shape[0],
                       weight.shape[1])(idx, weight)
    return out


# speedup vs baseline: 1.6452x; 1.0054x over previous
import functools

import jax
import jax.numpy as jnp
from jax import lax
from jax.experimental import pallas as pl
from jax.experimental.pallas import tpu as pltpu
from jax.experimental.pallas import tpu_sc as plsc

_CH = 400
_NBUF = 8


@functools.lru_cache(maxsize=None)
def _make_gather(B, S, V, D):
    info = plsc.get_sparse_core_info()
    nw = info.num_cores * info.num_subcores
    b_per_w = B // nw
    n_steps = b_per_w // _CH
    n_super = n_steps // _NBUF
    seq_per_ch = _CH // S
    mesh = plsc.VectorSubcoreMesh(core_axis_name="c", subcore_axis_name="s")

    @functools.partial(
        pl.kernel,
        mesh=mesh,
        out_type=jax.ShapeDtypeStruct((B // S, S, D), jnp.float32),
        scratch_types=[
            pltpu.VMEM((b_per_w,), jnp.int32),
            *[pltpu.VMEM((_CH, D), jnp.float32) for _ in range(_NBUF)],
            *[pltpu.SemaphoreType.DMA for _ in range(2 * _NBUF)],
        ],
        compiler_params=pltpu.CompilerParams(use_tc_tiling_on_sc=False),
    )
    def gather_kernel(idx_hbm, table_hbm, out_hbm, idx_all, *rest):
        rows = rest[:_NBUF]
        sems_g = rest[_NBUF:2 * _NBUF]
        sems_o = rest[2 * _NBUF:]
        wid = lax.axis_index("s") * info.num_cores + lax.axis_index("c")
        base = wid * b_per_w
        seq0 = wid * (b_per_w // S)
        pltpu.sync_copy(idx_hbm.at[pl.ds(base, b_per_w)], idx_all)

        def g_start(s, j):
            pltpu.async_copy(
                table_hbm.at[idx_all.at[pl.ds(s * _CH, _CH)]],
                rows[j], sems_g[j])

        def g_wait(j):
            pltpu.make_async_copy(
                table_hbm.at[pl.ds(0, _CH)], rows[j], sems_g[j]).wait()

        def o_start(s, j):
            for t in range(seq_per_ch):
                pltpu.async_copy(
                    rows[j].at[pl.ds(t * S, S)],
                    out_hbm.at[seq0 + s * seq_per_ch + t], sems_o[j])

        def o_wait(j):
            for t in range(seq_per_ch):
                pltpu.make_async_copy(
                    rows[j].at[pl.ds(0, S)], out_hbm.at[0],
                    sems_o[j]).wait()

        for j in range(_NBUF):
            g_start(j, j)

        def body(ss, carry):
            for j in range(_NBUF):
                g_wait(j)
                o_start(ss * _NBUF + j, j)

            @pl.when(ss + 1 < n_super)
            def _():
                for j in range(_NBUF):
                    o_wait(j)
                    g_start((ss + 1) * _NBUF + j, j)

            return carry

        lax.fori_loop(0, n_super, body, 0)
        for j in range(_NBUF):
            o_wait(j)

    return gather_kernel


def kernel(token_ids, weight):
    idx = token_ids.reshape(-1).astype(jnp.int32)
    n_seq, seq_len = token_ids.shape
    out = _make_gather(idx.shape[0], seq_len, weight.shape[0],
                       weight.shape[1])(idx, weight)
    return out
